# Initial kernel scaffold; baseline (speedup 1.0000x reference)
import jax
import jax.numpy as jnp
from jax.experimental import pallas as pl


def kernel(hidden, W_pred, emb_table):
    raise NotImplementedError("write your pallas kernel here")


# speedup vs baseline: 64.2113x; 64.2113x over previous
import functools

import jax
import jax.numpy as jnp
from jax import lax
from jax.experimental import pallas as pl
from jax.experimental.pallas import tpu as pltpu
from jax.experimental.pallas import tpu_sc as plsc

B, T, D = 1, 2048, 768
C = 100000
K = 16
G = 128
C_BLK = 512
N_CBLK = 196
C_PAD = N_CBLK * C_BLK
GPB = C_BLK // G
N_GRP = C_PAD // G
NEG = -3.0e38
NW = 32
TPW = T // NW
BIG_I = 2**30


def _matmul_kernel(h_ref, w_ref, logits_ref, gmax_ref):
    j = pl.program_id(0)
    acc = lax.dot_general(
        h_ref[...], w_ref[...],
        dimension_numbers=(((1,), (1,)), ((), ())),
        preferred_element_type=jnp.float32,
    )
    col = j * C_BLK + lax.broadcasted_iota(jnp.int32, (T, C_BLK), 1)
    acc = jnp.where(col < C, acc, NEG)
    logits_ref[...] = acc
    m = jnp.max(acc.reshape(T, GPB, G), axis=2)
    gmax_ref[...] = m.T.reshape(1, GPB, T)


def _logits_and_gmax(hidden2d, W_pred):
    return pl.pallas_call(
        _matmul_kernel,
        grid=(N_CBLK,),
        in_specs=[
            pl.BlockSpec((T, D), lambda j: (0, 0)),
            pl.BlockSpec((C_BLK, D), lambda j: (j, 0)),
        ],
        out_specs=[
            pl.BlockSpec((T, C_BLK), lambda j: (0, j)),
            pl.BlockSpec((1, GPB, T), lambda j: (j, 0, 0)),
        ],
        out_shape=[
            jax.ShapeDtypeStruct((T, C_PAD), jnp.float32),
            jax.ShapeDtypeStruct((N_CBLK, GPB, T), jnp.float32),
        ],
    )(hidden2d, W_pred)


T_BLK2 = 256


def _top_groups_kernel(g_ref, vals_ref, idxs_ref):
    g = g_ref[...]
    jb = lax.broadcasted_iota(jnp.int32, (N_CBLK, GPB, T_BLK2), 0)
    q = lax.broadcasted_iota(jnp.int32, (N_CBLK, GPB, T_BLK2), 1)
    gid = jb * GPB + q
    ms, ids = [], []
    for _ in range(K):
        m = jnp.max(jnp.max(g, axis=1), axis=0)
        mb = m[None, None, :]
        cand = jnp.where(g == mb, gid, BIG_I)
        idx = jnp.min(jnp.min(cand, axis=1), axis=0)
        g = jnp.where(gid == idx[None, None, :], NEG, g)
        ms.append(m)
        ids.append(idx)
    vals_ref[...] = jnp.stack(ms, axis=0)
    idxs_ref[...] = jnp.stack(ids, axis=0)


def _top_groups(gmax3):
    return pl.pallas_call(
        _top_groups_kernel,
        grid=(T // T_BLK2,),
        in_specs=[pl.BlockSpec((N_CBLK, GPB, T_BLK2), lambda i: (0, 0, i))],
        out_specs=[
            pl.BlockSpec((K, T_BLK2), lambda i: (0, i)),
            pl.BlockSpec((K, T_BLK2), lambda i: (0, i)),
        ],
        out_shape=[
            jax.ShapeDtypeStruct((K, T), jnp.float32),
            jax.ShapeDtypeStruct((K, T), jnp.int32),
        ],
    )(gmax3)


def _bcast_lane(vec, i):
    idx = jnp.broadcast_to(jnp.asarray(i, jnp.int32), (K,))
    dnums = lax.GatherDimensionNumbers(
        offset_dims=(), collapsed_slice_dims=(0,), start_index_map=(0,))
    return lax.gather(vec, idx[:, None], dnums, slice_sizes=(1,),
                      mode=lax.GatherScatterMode.PROMISE_IN_BOUNDS)


def _merge(run_v, run_i, ch_v, ch_i):
    ch_v, ch_i = plsc.sort_key_val(ch_v, ch_i)
    cond = (run_v > ch_v) | ((run_v == ch_v) & (run_i < ch_i))
    hi_v = jnp.where(cond, run_v, ch_v)
    hi_i = jnp.where(cond, run_i, ch_i)
    out_v, out_i = plsc.sort_key_val(hi_v, hi_i, descending=True)
    return out_v, out_i


def _sc_kernel(lrows, g16i, emb, feat, tki, tkl,
               gi_v, rows_v, erows_v, feat_v, tki_v, tkl_v, sem):
    cid = lax.axis_index("c")
    sid = lax.axis_index("s")
    wid = sid * 2 + cid
    lane = lax.broadcasted_iota(jnp.int32, (K,), 0)

    def token_body(i, _):
        t = wid * TPW + i
        pltpu.sync_copy(g16i.at[t], gi_v)
        groups = gi_v[...]
        row_ids = t * N_GRP + groups
        pltpu.async_copy(lrows.at[row_ids], rows_v, sem).wait()
        gbase = groups * G

        def outer(gslot, carry):
            run_v, run_i = carry
            base = _bcast_lane(gbase, gslot)

            def inner(cj, carry2):
                rv, ri = carry2
                ch = rows_v[gslot, pl.ds(cj * K, K)]
                ci = base + cj * K + lane
                return _merge(rv, ri, ch, ci)

            return lax.fori_loop(0, G // K, inner, (run_v, run_i))

        run_v = jnp.full((K,), NEG, jnp.float32)
        run_i = jnp.zeros((K,), jnp.int32)
        run_v, run_i = lax.fori_loop(0, K, outer, (run_v, run_i))

        tkl_v[...] = run_v
        tki_v[...] = run_i
        pltpu.sync_copy(tkl_v, tkl.at[t])
        pltpu.sync_copy(tki_v, tki.at[t])

        w = 1.0 / (1.0 + jnp.exp(-run_v))
        pltpu.async_copy(emb.at[run_i], erows_v, sem).wait()
        wbs = [_bcast_lane(w, r) for r in range(K)]

        def pool(d, _):
            acc = wbs[0] * erows_v[0, pl.ds(d * K, K)]
            for r in range(1, K):
                acc = acc + wbs[r] * erows_v[r, pl.ds(d * K, K)]
            feat_v[pl.ds(d * K, K)] = acc
            return 0

        lax.fori_loop(0, D // K, pool, 0)
        pltpu.sync_copy(feat_v, feat.at[t])
        return 0

    lax.fori_loop(0, TPW, token_body, 0)


def _sc_stage(lrows, g16i, emb):
    mesh = plsc.VectorSubcoreMesh(core_axis_name="c", subcore_axis_name="s")
    kfn = functools.partial(
        pl.kernel,
        mesh=mesh,
        compiler_params=pltpu.CompilerParams(needs_layout_passes=False),
        out_type=(
            jax.ShapeDtypeStruct((T, D), jnp.float32),
            jax.ShapeDtypeStruct((T, K), jnp.int32),
            jax.ShapeDtypeStruct((T, K), jnp.float32),
        ),
        scratch_types=[
            pltpu.VMEM((K,), jnp.int32),
            pltpu.VMEM((K, G), jnp.float32),
            pltpu.VMEM((K, D), jnp.float32),
            pltpu.VMEM((D,), jnp.float32),
            pltpu.VMEM((K,), jnp.int32),
            pltpu.VMEM((K,), jnp.float32),
            pltpu.SemaphoreType.DMA,
        ],
    )(_sc_kernel)
    return kfn(lrows, g16i, emb)


def kernel(hidden, W_pred, emb_table):
    hidden2d = hidden.reshape(T, D)
    logits, gmax3 = _logits_and_gmax(hidden2d, W_pred)
    _, g16i = _top_groups(gmax3)
    lrows = logits.reshape(T * N_GRP, G)
    feat, tki, tkl = _sc_stage(lrows, g16i.T, emb_table)
    return (feat.reshape(B, T, D),
            tki.reshape(B, T, K),
            tkl.reshape(B, T, K))


# speedup vs baseline: 64.3292x; 1.0018x over previous
import functools

import jax
import jax.numpy as jnp
from jax import lax
from jax.experimental import pallas as pl
from jax.experimental.pallas import tpu as pltpu
from jax.experimental.pallas import tpu_sc as plsc

B, T, D = 1, 2048, 768
C = 100000
K = 16
G = 128
C_BLK = 512
N_CBLK = 196
C_PAD = N_CBLK * C_BLK
GPB = C_BLK // G
N_GRP = C_PAD // G
NEG = -3.0e38
NW = 32
TPW = T // NW
BIG_I = 2**30


def _matmul_kernel(h_ref, w_ref, logits_ref, gmax_ref):
    j = pl.program_id(0)
    acc = lax.dot_general(
        h_ref[...], w_ref[...],
        dimension_numbers=(((1,), (1,)), ((), ())),
        preferred_element_type=jnp.float32,
    )
    col = j * C_BLK + lax.broadcasted_iota(jnp.int32, (T, C_BLK), 1)
    acc = jnp.where(col < C, acc, NEG)
    logits_ref[...] = acc
    m = jnp.max(acc.reshape(T, GPB, G), axis=2)
    gmax_ref[...] = m.T.reshape(1, GPB, T)


def _logits_and_gmax(hidden2d, W_pred):
    return pl.pallas_call(
        _matmul_kernel,
        grid=(N_CBLK,),
        in_specs=[
            pl.BlockSpec((T, D), lambda j: (0, 0)),
            pl.BlockSpec((C_BLK, D), lambda j: (j, 0)),
        ],
        out_specs=[
            pl.BlockSpec((T, C_BLK), lambda j: (0, j)),
            pl.BlockSpec((1, GPB, T), lambda j: (j, 0, 0)),
        ],
        out_shape=[
            jax.ShapeDtypeStruct((T, C_PAD), jnp.float32),
            jax.ShapeDtypeStruct((N_CBLK, GPB, T), jnp.float32),
        ],
        compiler_params=pltpu.CompilerParams(
            dimension_semantics=("parallel",)),
    )(hidden2d, W_pred)


T_BLK2 = 256


def _top_groups_kernel(g_ref, vals_ref, idxs_ref):
    g = g_ref[...]
    jb = lax.broadcasted_iota(jnp.int32, (N_CBLK, GPB, T_BLK2), 0)
    q = lax.broadcasted_iota(jnp.int32, (N_CBLK, GPB, T_BLK2), 1)
    gid = jb * GPB + q
    ms, ids = [], []
    for _ in range(K):
        m = jnp.max(jnp.max(g, axis=1), axis=0)
        mb = m[None, None, :]
        cand = jnp.where(g == mb, gid, BIG_I)
        idx = jnp.min(jnp.min(cand, axis=1), axis=0)
        g = jnp.where(gid == idx[None, None, :], NEG, g)
        ms.append(m)
        ids.append(idx)
    vals_ref[...] = jnp.stack(ms, axis=0)
    idxs_ref[...] = jnp.stack(ids, axis=0)


def _top_groups(gmax3):
    return pl.pallas_call(
        _top_groups_kernel,
        grid=(T // T_BLK2,),
        in_specs=[pl.BlockSpec((N_CBLK, GPB, T_BLK2), lambda i: (0, 0, i))],
        out_specs=[
            pl.BlockSpec((K, T_BLK2), lambda i: (0, i)),
            pl.BlockSpec((K, T_BLK2), lambda i: (0, i)),
        ],
        out_shape=[
            jax.ShapeDtypeStruct((K, T), jnp.float32),
            jax.ShapeDtypeStruct((K, T), jnp.int32),
        ],
        compiler_params=pltpu.CompilerParams(
            dimension_semantics=("parallel",)),
    )(gmax3)


def _bcast_lane(vec, i):
    idx = jnp.broadcast_to(jnp.asarray(i, jnp.int32), (K,))
    dnums = lax.GatherDimensionNumbers(
        offset_dims=(), collapsed_slice_dims=(0,), start_index_map=(0,))
    return lax.gather(vec, idx[:, None], dnums, slice_sizes=(1,),
                      mode=lax.GatherScatterMode.PROMISE_IN_BOUNDS)


def _merge(run_v, run_i, ch_v, ch_i):
    ch_v, ch_i = plsc.sort_key_val(ch_v, ch_i)
    cond = (run_v > ch_v) | ((run_v == ch_v) & (run_i < ch_i))
    hi_v = jnp.where(cond, run_v, ch_v)
    hi_i = jnp.where(cond, run_i, ch_i)
    out_v, out_i = plsc.sort_key_val(hi_v, hi_i, descending=True)
    return out_v, out_i


def _sc_kernel(lrows, g16i, emb, feat, tki, tkl,
               gi_v, rows_v, erows_v, feat_v, tki_v, tkl_v, sem):
    cid = lax.axis_index("c")
    sid = lax.axis_index("s")
    wid = sid * 2 + cid
    lane = lax.broadcasted_iota(jnp.int32, (K,), 0)

    def token_body(i, _):
        t = wid * TPW + i
        pltpu.sync_copy(g16i.at[t], gi_v)
        groups = gi_v[...]
        row_ids = t * N_GRP + groups
        pltpu.async_copy(lrows.at[row_ids], rows_v, sem).wait()
        gbase = groups * G

        def outer(gslot, carry):
            run_v, run_i = carry
            base = _bcast_lane(gbase, gslot)

            def inner(cj, carry2):
                rv, ri = carry2
                ch = rows_v[gslot, pl.ds(cj * K, K)]
                ci = base + cj * K + lane
                return _merge(rv, ri, ch, ci)

            return lax.fori_loop(0, G // K, inner, (run_v, run_i))

        run_v = jnp.full((K,), NEG, jnp.float32)
        run_i = jnp.zeros((K,), jnp.int32)
        run_v, run_i = lax.fori_loop(0, K, outer, (run_v, run_i))

        tkl_v[...] = run_v
        tki_v[...] = run_i
        pltpu.sync_copy(tkl_v, tkl.at[t])
        pltpu.sync_copy(tki_v, tki.at[t])

        w = 1.0 / (1.0 + jnp.exp(-run_v))
        pltpu.async_copy(emb.at[run_i], erows_v, sem).wait()
        wbs = [_bcast_lane(w, r) for r in range(K)]

        def pool(d, _):
            acc = wbs[0] * erows_v[0, pl.ds(d * K, K)]
            for r in range(1, K):
                acc = acc + wbs[r] * erows_v[r, pl.ds(d * K, K)]
            feat_v[pl.ds(d * K, K)] = acc
            return 0

        lax.fori_loop(0, D // K, pool, 0)
        pltpu.sync_copy(feat_v, feat.at[t])
        return 0

    lax.fori_loop(0, TPW, token_body, 0)


def _sc_stage(lrows, g16i, emb):
    mesh = plsc.VectorSubcoreMesh(core_axis_name="c", subcore_axis_name="s")
    kfn = functools.partial(
        pl.kernel,
        mesh=mesh,
        compiler_params=pltpu.CompilerParams(needs_layout_passes=False),
        out_type=(
            jax.ShapeDtypeStruct((T, D), jnp.float32),
            jax.ShapeDtypeStruct((T, K), jnp.int32),
            jax.ShapeDtypeStruct((T, K), jnp.float32),
        ),
        scratch_types=[
            pltpu.VMEM((K,), jnp.int32),
            pltpu.VMEM((K, G), jnp.float32),
            pltpu.VMEM((K, D), jnp.float32),
            pltpu.VMEM((D,), jnp.float32),
            pltpu.VMEM((K,), jnp.int32),
            pltpu.VMEM((K,), jnp.float32),
            pltpu.SemaphoreType.DMA,
        ],
    )(_sc_kernel)
    return kfn(lrows, g16i, emb)


def kernel(hidden, W_pred, emb_table):
    hidden2d = hidden.reshape(T, D)
    logits, gmax3 = _logits_and_gmax(hidden2d, W_pred)
    _, g16i = _top_groups(gmax3)
    lrows = logits.reshape(T * N_GRP, G)
    feat, tki, tkl = _sc_stage(lrows, g16i.T, emb_table)
    return (feat.reshape(B, T, D),
            tki.reshape(B, T, K),
            tkl.reshape(B, T, K))


# speedup vs baseline: 65.6286x; 1.0202x over previous
import functools

import jax
import jax.numpy as jnp
from jax import lax
from jax.experimental import pallas as pl
from jax.experimental.pallas import tpu as pltpu
from jax.experimental.pallas import tpu_sc as plsc

B, T, D = 1, 2048, 768
C = 100000
K = 16
G = 128
C_BLK = 512
N_CBLK = 196
C_PAD = N_CBLK * C_BLK
GPB = C_BLK // G
N_GRP = C_PAD // G
NEG = -3.0e38
NW = 32
TPW = T // NW
BIG_I = 2**30


def _matmul_kernel(h_ref, w_ref, logits_ref, gmax_ref):
    j = pl.program_id(0)
    acc = lax.dot_general(
        h_ref[...], w_ref[...],
        dimension_numbers=(((1,), (1,)), ((), ())),
        preferred_element_type=jnp.float32,
    )
    col = j * C_BLK + lax.broadcasted_iota(jnp.int32, (T, C_BLK), 1)
    acc = jnp.where(col < C, acc, NEG)
    logits_ref[...] = acc
    m = jnp.max(acc.reshape(T, GPB, G), axis=2)
    gmax_ref[...] = m.T.reshape(1, GPB, T)


def _logits_and_gmax(hidden2d, W_pred):
    return pl.pallas_call(
        _matmul_kernel,
        grid=(N_CBLK,),
        in_specs=[
            pl.BlockSpec((T, D), lambda j: (0, 0)),
            pl.BlockSpec((C_BLK, D), lambda j: (j, 0)),
        ],
        out_specs=[
            pl.BlockSpec((T, C_BLK), lambda j: (0, j)),
            pl.BlockSpec((1, GPB, T), lambda j: (j, 0, 0)),
        ],
        out_shape=[
            jax.ShapeDtypeStruct((T, C_PAD), jnp.float32),
            jax.ShapeDtypeStruct((N_CBLK, GPB, T), jnp.float32),
        ],
        compiler_params=pltpu.CompilerParams(
            dimension_semantics=("parallel",)),
    )(hidden2d, W_pred)


T_BLK2 = 256


def _top_groups_kernel(g_ref, vals_ref, idxs_ref):
    g = g_ref[...]
    jb = lax.broadcasted_iota(jnp.int32, (N_CBLK, GPB, T_BLK2), 0)
    q = lax.broadcasted_iota(jnp.int32, (N_CBLK, GPB, T_BLK2), 1)
    gid = jb * GPB + q
    ms, ids = [], []
    for _ in range(K):
        m = jnp.max(jnp.max(g, axis=1), axis=0)
        mb = m[None, None, :]
        cand = jnp.where(g == mb, gid, BIG_I)
        idx = jnp.min(jnp.min(cand, axis=1), axis=0)
        g = jnp.where(gid == idx[None, None, :], NEG, g)
        ms.append(m)
        ids.append(idx)
    vals_ref[...] = jnp.stack(ms, axis=0)
    idxs_ref[...] = jnp.stack(ids, axis=0)


def _top_groups(gmax3):
    return pl.pallas_call(
        _top_groups_kernel,
        grid=(T // T_BLK2,),
        in_specs=[pl.BlockSpec((N_CBLK, GPB, T_BLK2), lambda i: (0, 0, i))],
        out_specs=[
            pl.BlockSpec((K, T_BLK2), lambda i: (0, i)),
            pl.BlockSpec((K, T_BLK2), lambda i: (0, i)),
        ],
        out_shape=[
            jax.ShapeDtypeStruct((K, T), jnp.float32),
            jax.ShapeDtypeStruct((K, T), jnp.int32),
        ],
        compiler_params=pltpu.CompilerParams(
            dimension_semantics=("parallel",)),
    )(gmax3)


def _bcast_lane(vec, i):
    idx = jnp.broadcast_to(jnp.asarray(i, jnp.int32), (K,))
    dnums = lax.GatherDimensionNumbers(
        offset_dims=(), collapsed_slice_dims=(0,), start_index_map=(0,))
    return lax.gather(vec, idx[:, None], dnums, slice_sizes=(1,),
                      mode=lax.GatherScatterMode.PROMISE_IN_BOUNDS)


def _merge(run_v, run_i, ch_v, ch_i):
    ch_v, ch_i = plsc.sort_key_val(ch_v, ch_i)
    cond = (run_v > ch_v) | ((run_v == ch_v) & (run_i < ch_i))
    hi_v = jnp.where(cond, run_v, ch_v)
    hi_i = jnp.where(cond, run_i, ch_i)
    out_v, out_i = plsc.sort_key_val(hi_v, hi_i, descending=True)
    return out_v, out_i


def _sc_kernel(lrows, g16i, emb, feat, tki, tkl,
               gidx_s, rows_v, erows_v, feats_s, tki_s, tkl_s, sem):
    cid = lax.axis_index("c")
    sid = lax.axis_index("s")
    wid = sid * 2 + cid
    lane = lax.broadcasted_iota(jnp.int32, (K,), 0)

    t0 = wid * TPW
    pltpu.sync_copy(g16i.at[pl.ds(t0, TPW)], gidx_s)

    def token_body(i, _):
        t = t0 + i
        groups = gidx_s[i]
        row_ids = t * N_GRP + groups
        pltpu.async_copy(lrows.at[row_ids], rows_v, sem).wait()
        gbase = groups * G

        def outer(gslot, carry):
            run_v, run_i = carry
            base = _bcast_lane(gbase, gslot)

            def inner(cj, carry2):
                rv, ri = carry2
                ch = rows_v[gslot, pl.ds(cj * K, K)]
                ci = base + cj * K + lane
                return _merge(rv, ri, ch, ci)

            return lax.fori_loop(0, G // K, inner, (run_v, run_i))

        run_v = jnp.full((K,), NEG, jnp.float32)
        run_i = jnp.zeros((K,), jnp.int32)
        run_v, run_i = lax.fori_loop(0, K, outer, (run_v, run_i))

        tkl_s[i] = run_v
        tki_s[i] = run_i

        w = 1.0 / (1.0 + jnp.exp(-run_v))
        pltpu.async_copy(emb.at[run_i], erows_v, sem).wait()
        wbs = [_bcast_lane(w, r) for r in range(K)]

        def pool(d, _):
            acc = wbs[0] * erows_v[0, pl.ds(d * K, K)]
            for r in range(1, K):
                acc = acc + wbs[r] * erows_v[r, pl.ds(d * K, K)]
            feats_s[i, pl.ds(d * K, K)] = acc
            return 0

        lax.fori_loop(0, D // K, pool, 0)
        return 0

    lax.fori_loop(0, TPW, token_body, 0)

    pltpu.sync_copy(tkl_s, tkl.at[pl.ds(t0, TPW)])
    pltpu.sync_copy(tki_s, tki.at[pl.ds(t0, TPW)])
    pltpu.sync_copy(feats_s, feat.at[pl.ds(t0, TPW)])


def _sc_stage(lrows, g16i, emb):
    mesh = plsc.VectorSubcoreMesh(core_axis_name="c", subcore_axis_name="s")
    kfn = functools.partial(
        pl.kernel,
        mesh=mesh,
        compiler_params=pltpu.CompilerParams(needs_layout_passes=False),
        out_type=(
            jax.ShapeDtypeStruct((T, D), jnp.float32),
            jax.ShapeDtypeStruct((T, K), jnp.int32),
            jax.ShapeDtypeStruct((T, K), jnp.float32),
        ),
        scratch_types=[
            pltpu.VMEM((TPW, K), jnp.int32),
            pltpu.VMEM((K, G), jnp.float32),
            pltpu.VMEM((K, D), jnp.float32),
            pltpu.VMEM((TPW, D), jnp.float32),
            pltpu.VMEM((TPW, K), jnp.int32),
            pltpu.VMEM((TPW, K), jnp.float32),
            pltpu.SemaphoreType.DMA,
        ],
    )(_sc_kernel)
    return kfn(lrows, g16i, emb)


def kernel(hidden, W_pred, emb_table):
    hidden2d = hidden.reshape(T, D)
    logits, gmax3 = _logits_and_gmax(hidden2d, W_pred)
    _, g16i = _top_groups(gmax3)
    lrows = logits.reshape(T * N_GRP, G)
    feat, tki, tkl = _sc_stage(lrows, g16i.T, emb_table)
    return (feat.reshape(B, T, D),
            tki.reshape(B, T, K),
            tkl.reshape(B, T, K))
